# baseline (device time: 458294 ns/iter reference)
import jax
import jax.numpy as jnp
from jax import lax
from jax.experimental import pallas as pl
from jax.experimental.pallas import tpu as pltpu

MH = 4096
D = 4096
HALF = MH // 2
AMAX = 256
W = [256] * 7 + [128, 64, 32, 16, 8, 8]
assert sum(W) == HALF
OFF = [sum(W[:i]) for i in range(len(W))]
NCH = len(W)
EPS = 1e-6


def kernel(partial, gamma):
    gamma2d = gamma.reshape(1, D)

    def body(p_ref, g_ref, out_ref, recv_y, a, o, local_sems,
             y_send_sems, y_recv_sems, x_send_sems, x_recv_sems):
        my_x = lax.axis_index("x")
        my_y = lax.axis_index("y")
        my_z = lax.axis_index("z")
        y_peer = (my_x, 1 - my_y, my_z)
        x_peer = (1 - my_x, my_y, my_z)

        barrier = pltpu.get_barrier_semaphore()
        for peer in (y_peer, x_peer):
            pl.semaphore_signal(barrier, inc=1, device_id=peer,
                                device_id_type=pl.DeviceIdType.MESH)
        pl.semaphore_wait(barrier, 2)

        y_rdmas = []
        for c in range(NCH):
            rdma = pltpu.make_async_remote_copy(
                src_ref=p_ref.at[
                    0, pl.ds((1 - my_y) * MH + my_x * HALF + OFF[c], W[c]), :],
                dst_ref=recv_y.at[pl.ds(OFF[c], W[c]), :],
                send_sem=y_send_sems.at[c],
                recv_sem=y_recv_sems.at[c],
                device_id=y_peer,
                device_id_type=pl.DeviceIdType.MESH,
            )
            rdma.start()
            y_rdmas.append(rdma)

        x_in = []
        for c in range(NCH):
            x_in.append(pltpu.make_async_remote_copy(
                src_ref=o.at[0, pl.ds(0, W[c]), :],
                dst_ref=out_ref.at[
                    pl.ds((1 - my_x) * HALF + OFF[c], W[c]), :],
                send_sem=x_send_sems.at[c],
                recv_sem=x_recv_sems.at[c],
                device_id=x_peer,
                device_id_type=pl.DeviceIdType.MESH,
            ))

        def a_load(c):
            cp = pltpu.make_async_copy(
                p_ref.at[
                    0, pl.ds(my_y * MH + my_x * HALF + OFF[c], W[c]), :],
                a.at[c % 2, pl.ds(0, W[c]), :], local_sems.at[c % 2])
            cp.start()
            return cp

        a_loads = {0: a_load(0)}
        x_out = []
        o_stores = []
        for c in range(NCH):
            r = my_x * HALF + OFF[c]
            if c + 1 < NCH:
                a_loads[c + 1] = a_load(c + 1)
            if c >= 2:
                x_out[c - 2].wait_send()
                o_stores[c - 2].wait()
            y_rdmas[c].wait_recv()
            a_loads[c].wait()
            y = a[c % 2, :W[c], :] + recv_y[OFF[c]:OFF[c] + W[c], :]
            ms = jnp.mean(y * y, axis=-1, keepdims=True)
            o[c % 2, :W[c], :] = y * lax.rsqrt(ms + EPS) * g_ref[...]
            rdma_x = pltpu.make_async_remote_copy(
                src_ref=o.at[c % 2, pl.ds(0, W[c]), :],
                dst_ref=out_ref.at[pl.ds(r, W[c]), :],
                send_sem=x_send_sems.at[c],
                recv_sem=x_recv_sems.at[c],
                device_id=x_peer,
                device_id_type=pl.DeviceIdType.MESH,
            )
            rdma_x.start()
            x_out.append(rdma_x)
            cp_o = pltpu.make_async_copy(
                o.at[c % 2, pl.ds(0, W[c]), :],
                out_ref.at[pl.ds(r, W[c]), :],
                local_sems.at[3 + (c % 2)])
            cp_o.start()
            o_stores.append(cp_o)

        for c in range(NCH):
            x_in[c].wait_recv()
            y_rdmas[c].wait_send()
        for c in range(max(NCH - 2, 0), NCH):
            x_out[c].wait_send()
            o_stores[c].wait()

    return pl.pallas_call(
        body,
        out_shape=jax.ShapeDtypeStruct((MH, D), jnp.float32),
        in_specs=[
            pl.BlockSpec(memory_space=pl.ANY),
            pl.BlockSpec(memory_space=pltpu.MemorySpace.VMEM),
        ],
        out_specs=pl.BlockSpec(memory_space=pl.ANY),
        scratch_shapes=[
            pltpu.VMEM((HALF, D), jnp.float32),
            pltpu.VMEM((2, AMAX, D), jnp.float32),
            pltpu.VMEM((2, AMAX, D), jnp.float32),
            pltpu.SemaphoreType.DMA((5,)),
            pltpu.SemaphoreType.DMA((NCH,)),
            pltpu.SemaphoreType.DMA((NCH,)),
            pltpu.SemaphoreType.DMA((NCH,)),
            pltpu.SemaphoreType.DMA((NCH,)),
        ],
        compiler_params=pltpu.CompilerParams(
            collective_id=0,
            vmem_limit_bytes=100 * 1024 * 1024,
        ),
    )(partial, gamma2d)


# device time: 348578 ns/iter; 1.3148x vs baseline; 1.3148x over previous
import jax
import jax.numpy as jnp
from jax import lax
from jax.experimental import pallas as pl
from jax.experimental.pallas import tpu as pltpu

MH = 4096
D = 4096
QR = MH // 4
CHUNK = 128
FH = CHUNK // 2
NCH = QR // CHUNK
LAG = 2
EPS = 1e-6


def kernel(partial, gamma):
    gamma2d = gamma.reshape(1, D)

    def body(p_ref, g_ref, out_ref, recv_y, a, b, o, local_sems,
             y_send_sems, y_recv_sems, xd_send_sems, xd_recv_sems,
             zd_send_sems, zd_recv_sems, xf_send_sems, xf_recv_sems,
             zf_send_sems, zf_recv_sems):
        my_x = lax.axis_index("x")
        my_y = lax.axis_index("y")
        my_z = lax.axis_index("z")
        my_zp = lax.rem(my_z, 2)
        y_peer = (my_x, 1 - my_y, my_z)
        x_peer = (1 - my_x, my_y, my_z)
        z_buddy = (my_x, my_y, my_z + 1 - 2 * my_zp)

        qm = (2 * my_x + my_zp) * QR
        qx = (2 * (1 - my_x) + my_zp) * QR
        qz = (2 * my_x + (1 - my_zp)) * QR
        qd = (2 * (1 - my_x) + (1 - my_zp)) * QR

        barrier = pltpu.get_barrier_semaphore()
        for peer in (y_peer, x_peer, z_buddy):
            pl.semaphore_signal(barrier, inc=1, device_id=peer,
                                device_id_type=pl.DeviceIdType.MESH)
        pl.semaphore_wait(barrier, 3)

        y_rdmas = []
        for c in range(NCH):
            rdma = pltpu.make_async_remote_copy(
                src_ref=p_ref.at[
                    0, pl.ds((1 - my_y) * MH + qm + c * CHUNK, CHUNK), :],
                dst_ref=recv_y.at[pl.ds(c * CHUNK, CHUNK), :],
                send_sem=y_send_sems.at[c],
                recv_sem=y_recv_sems.at[c],
                device_id=y_peer,
                device_id_type=pl.DeviceIdType.MESH,
            )
            rdma.start()
            y_rdmas.append(rdma)

        xd_in = [pltpu.make_async_remote_copy(
            src_ref=o.at[0],
            dst_ref=out_ref.at[pl.ds(qx + c * CHUNK, CHUNK), :],
            send_sem=xd_send_sems.at[c], recv_sem=xd_recv_sems.at[c],
            device_id=x_peer, device_id_type=pl.DeviceIdType.MESH,
        ) for c in range(NCH)]
        zd_in = [pltpu.make_async_remote_copy(
            src_ref=o.at[0],
            dst_ref=out_ref.at[pl.ds(qz + c * CHUNK, CHUNK), :],
            send_sem=zd_send_sems.at[c], recv_sem=zd_recv_sems.at[c],
            device_id=z_buddy, device_id_type=pl.DeviceIdType.MESH,
        ) for c in range(NCH)]
        zf_in = [pltpu.make_async_remote_copy(
            src_ref=o.at[0, pl.ds(0, FH), :],
            dst_ref=out_ref.at[pl.ds(qd + c * CHUNK, FH), :],
            send_sem=zf_send_sems.at[c], recv_sem=zf_recv_sems.at[c],
            device_id=z_buddy, device_id_type=pl.DeviceIdType.MESH,
        ) for c in range(NCH)]
        xf_in = [pltpu.make_async_remote_copy(
            src_ref=o.at[0, pl.ds(0, FH), :],
            dst_ref=out_ref.at[pl.ds(qd + c * CHUNK + FH, FH), :],
            send_sem=xf_send_sems.at[c], recv_sem=xf_recv_sems.at[c],
            device_id=x_peer, device_id_type=pl.DeviceIdType.MESH,
        ) for c in range(NCH)]

        def a_load(c):
            cp = pltpu.make_async_copy(
                p_ref.at[0, pl.ds(my_y * MH + qm + c * CHUNK, CHUNK), :],
                a.at[c % 2], local_sems.at[c % 2])
            cp.start()
            return cp

        def forward(c):
            xd_in[c].wait_recv()
            zf = pltpu.make_async_remote_copy(
                src_ref=out_ref.at[pl.ds(qx + c * CHUNK, FH), :],
                dst_ref=out_ref.at[pl.ds(qx + c * CHUNK, FH), :],
                send_sem=zf_send_sems.at[c], recv_sem=zf_recv_sems.at[c],
                device_id=z_buddy, device_id_type=pl.DeviceIdType.MESH,
            )
            zf.start()
            zd_in[c].wait_recv()
            xf = pltpu.make_async_remote_copy(
                src_ref=out_ref.at[pl.ds(qz + c * CHUNK + FH, FH), :],
                dst_ref=out_ref.at[pl.ds(qz + c * CHUNK + FH, FH), :],
                send_sem=xf_send_sems.at[c], recv_sem=xf_recv_sems.at[c],
                device_id=x_peer, device_id_type=pl.DeviceIdType.MESH,
            )
            xf.start()
            return zf, xf

        a_loads = {0: a_load(0)}
        xd_out = []
        zd_out = []
        fwds = []
        o_stores = []
        for c in range(NCH):
            r = qm + c * CHUNK
            if c + 1 < NCH:
                a_loads[c + 1] = a_load(c + 1)
            y_rdmas[c].wait_recv()
            cp_b = pltpu.make_async_copy(
                recv_y.at[pl.ds(c * CHUNK, CHUNK), :], b, local_sems.at[2])
            cp_b.start()
            if c >= 2:
                xd_out[c - 2].wait_send()
                zd_out[c - 2].wait_send()
                o_stores[c - 2].wait()
            a_loads[c].wait()
            cp_b.wait()
            y = a[c % 2] + b[...]
            ms = jnp.mean(y * y, axis=-1, keepdims=True)
            o[c % 2] = y * lax.rsqrt(ms + EPS) * g_ref[...]
            for peer, sends, recvs, lst in (
                (x_peer, xd_send_sems, xd_recv_sems, xd_out),
                (z_buddy, zd_send_sems, zd_recv_sems, zd_out),
            ):
                rd = pltpu.make_async_remote_copy(
                    src_ref=o.at[c % 2],
                    dst_ref=out_ref.at[pl.ds(r, CHUNK), :],
                    send_sem=sends.at[c], recv_sem=recvs.at[c],
                    device_id=peer, device_id_type=pl.DeviceIdType.MESH,
                )
                rd.start()
                lst.append(rd)
            cp_o = pltpu.make_async_copy(
                o.at[c % 2], out_ref.at[pl.ds(r, CHUNK), :],
                local_sems.at[3 + (c % 2)])
            cp_o.start()
            o_stores.append(cp_o)
            if c >= LAG:
                fwds.append(forward(c - LAG))

        for c in range(max(NCH - LAG, 0), NCH):
            fwds.append(forward(c))
        for c in range(NCH):
            zf_in[c].wait_recv()
            xf_in[c].wait_recv()
            y_rdmas[c].wait_send()
        for zf, xf in fwds:
            zf.wait_send()
            xf.wait_send()
        for c in range(max(NCH - 2, 0), NCH):
            xd_out[c].wait_send()
            zd_out[c].wait_send()
            o_stores[c].wait()

    out, _ = pl.pallas_call(
        body,
        out_shape=(
            jax.ShapeDtypeStruct((MH, D), jnp.float32),
            jax.ShapeDtypeStruct((QR, D), jnp.float32),
        ),
        in_specs=[
            pl.BlockSpec(memory_space=pl.ANY),
            pl.BlockSpec(memory_space=pltpu.MemorySpace.VMEM),
        ],
        out_specs=(
            pl.BlockSpec(memory_space=pl.ANY),
            pl.BlockSpec(memory_space=pl.ANY),
        ),
        scratch_shapes=[
            pltpu.VMEM((2, CHUNK, D), jnp.float32),
            pltpu.VMEM((CHUNK, D), jnp.float32),
            pltpu.VMEM((2, CHUNK, D), jnp.float32),
            pltpu.SemaphoreType.DMA((5,)),
            pltpu.SemaphoreType.DMA((NCH,)),
            pltpu.SemaphoreType.DMA((NCH,)),
            pltpu.SemaphoreType.DMA((NCH,)),
            pltpu.SemaphoreType.DMA((NCH,)),
            pltpu.SemaphoreType.DMA((NCH,)),
            pltpu.SemaphoreType.DMA((NCH,)),
            pltpu.SemaphoreType.DMA((NCH,)),
            pltpu.SemaphoreType.DMA((NCH,)),
            pltpu.SemaphoreType.DMA((NCH,)),
            pltpu.SemaphoreType.DMA((NCH,)),
        ],
        compiler_params=pltpu.CompilerParams(collective_id=0),
    )(partial, gamma2d)
    return out
